# baseline (speedup 1.0000x reference)
import functools

import jax
import jax.numpy as jnp
import jax.scipy.linalg as jsl
from jax import lax
from jax.experimental import pallas as pl
from jax.experimental.pallas import tpu as pltpu
from jax.experimental.pallas import tpu_sc as plsc

N = 10000
E = 160000
D = 256
DE = 16

NC = 2
NS = 16
NW = NC * NS
B = 128
NB = E // B
RPS = N // NS


_NBLK = 1000
_EBLK = 3200


def _nodeproj_body(x_ref, w_ref, q_ref, pv_ref):
    r = jnp.dot(x_ref[...], w_ref[...], preferred_element_type=jnp.float32)
    q_ref[0, :, :] = r[:, 0:128]
    q_ref[1, :, :] = r[:, 128:256]
    pv_ref[:, 0:32] = r[:, 256:288]
    pv_ref[:, 32:128] = jnp.zeros_like(r[:, 32:128])


def _node_proj(x, wcat):
    grid = (N // _NBLK,)
    q, pv = pl.pallas_call(
        _nodeproj_body,
        grid=grid,
        in_specs=[
            pl.BlockSpec((_NBLK, D), lambda i: (i, 0)),
            pl.BlockSpec((D, 288), lambda i: (0, 0)),
        ],
        out_specs=[
            pl.BlockSpec((2, _NBLK, 128), lambda i: (0, i, 0)),
            pl.BlockSpec((_NBLK, 128), lambda i: (i, 0)),
        ],
        out_shape=[
            jax.ShapeDtypeStruct((2, N, 128), jnp.float32),
            jax.ShapeDtypeStruct((N, 128), jnp.float32),
        ],
    )(x, wcat)
    return q, pv


def _edgeproj_body(e_ref, w_ref, b_ref, o_ref):
    o_ref[...] = (
        jnp.dot(e_ref[...], w_ref[...], preferred_element_type=jnp.float32)
        + b_ref[...]
    )


def _edge_proj(eaw, wee8, bew):
    wblk = _EBLK // 8
    return pl.pallas_call(
        _edgeproj_body,
        grid=(E // _EBLK,),
        in_specs=[
            pl.BlockSpec((wblk, 128), lambda i: (i, 0)),
            pl.BlockSpec((128, 128), lambda i: (0, 0)),
            pl.BlockSpec((1, 128), lambda i: (0, 0)),
        ],
        out_specs=pl.BlockSpec((wblk, 128), lambda i: (i, 0)),
        out_shape=jax.ShapeDtypeStruct((E // 8, 128), jnp.float32),
    )(eaw, wee8, bew)


def _tproj_body(avg, e_ref, ep_ref, wbig_ref, bmw_ref, wee_ref, be_ref,
                t_ref, en_ref):
    e = e_ref[...]
    t = jnp.dot(e, wbig_ref[...], preferred_element_type=jnp.float32) + bmw_ref[...]
    t_ref[0, :, :] = t[:, 0:1024]
    t_ref[1, :, :] = t[:, 1024:2048]
    ea = 0.5 * (e + ep_ref[...]) if avg else e
    en_ref[...] = (
        jnp.dot(ea, wee_ref[...], preferred_element_type=jnp.float32)
        + be_ref[...]
    )


def _t_proj(e_new, e_prev, wbig, bmw, wee8, bew, avg):
    wblk = _EBLK // 8
    t, en = pl.pallas_call(
        functools.partial(_tproj_body, avg),
        grid=(E // _EBLK,),
        in_specs=[
            pl.BlockSpec((wblk, 128), lambda i: (i, 0)),
            pl.BlockSpec((wblk, 128), lambda i: (i, 0)),
            pl.BlockSpec((128, 2048), lambda i: (0, 0)),
            pl.BlockSpec((1, 2048), lambda i: (0, 0)),
            pl.BlockSpec((128, 128), lambda i: (0, 0)),
            pl.BlockSpec((1, 128), lambda i: (0, 0)),
        ],
        out_specs=[
            pl.BlockSpec((2, wblk, 1024), lambda i: (0, i, 0)),
            pl.BlockSpec((wblk, 128), lambda i: (i, 0)),
        ],
        out_shape=[
            jax.ShapeDtypeStruct((2, E // 8, 1024), jnp.float32),
            jax.ShapeDtypeStruct((E // 8, 128), jnp.float32),
        ],
    )(e_new, e_prev, wbig, bmw, wee8, bew)
    return t, en


def _nodeup_body(avg, ncols, x_ref, a0_ref, a1_ref, wna_ref, wnb0_ref,
                 wnb1_ref, bn_ref, wcat_ref, *out_refs):
    x = x_ref[...]
    r = jnp.dot(x, wna_ref[...], preferred_element_type=jnp.float32)
    r = r + jnp.dot(a0_ref[0], wnb0_ref[...], preferred_element_type=jnp.float32)
    r = r + jnp.dot(a1_ref[0], wnb1_ref[...], preferred_element_type=jnp.float32)
    r = jnp.maximum(r + bn_ref[...], 0.0)
    if avg:
        r = 0.5 * (x + r)
    p = jnp.dot(r, wcat_ref[...], preferred_element_type=jnp.float32)
    if ncols == 288:
        xo_ref, q_ref, pv_ref = out_refs
        xo_ref[...] = r
        q_ref[0, :, :] = p[:, 0:128]
        q_ref[1, :, :] = p[:, 128:256]
        pv_ref[:, 0:32] = p[:, 256:288]
        pv_ref[:, 32:128] = jnp.zeros_like(p[:, 32:128])
    else:
        pv_ref, = out_refs
        pv_ref[:, 0:32] = p[:, 0:32]
        pv_ref[:, 32:128] = jnp.zeros((p.shape[0], 96), jnp.float32)


def _node_update(x, agg, wna, wnb0, wnb1, bn, wcat, avg):
    ncols = wcat.shape[1]
    if ncols == 288:
        out_specs = [
            pl.BlockSpec((_NBLK, D), lambda i: (i, 0)),
            pl.BlockSpec((2, _NBLK, 128), lambda i: (0, i, 0)),
            pl.BlockSpec((_NBLK, 128), lambda i: (i, 0)),
        ]
        out_shape = [
            jax.ShapeDtypeStruct((N, D), jnp.float32),
            jax.ShapeDtypeStruct((2, N, 128), jnp.float32),
            jax.ShapeDtypeStruct((N, 128), jnp.float32),
        ]
    else:
        out_specs = pl.BlockSpec((_NBLK, 128), lambda i: (i, 0))
        out_shape = jax.ShapeDtypeStruct((N, 128), jnp.float32)
    return pl.pallas_call(
        functools.partial(_nodeup_body, avg, ncols),
        grid=(N // _NBLK,),
        in_specs=[
            pl.BlockSpec((_NBLK, D), lambda i: (i, 0)),
            pl.BlockSpec((1, _NBLK, 128), lambda i: (0, i, 0)),
            pl.BlockSpec((1, _NBLK, 128), lambda i: (1, i, 0)),
            pl.BlockSpec((D, D), lambda i: (0, 0)),
            pl.BlockSpec((128, D), lambda i: (0, 0)),
            pl.BlockSpec((128, D), lambda i: (0, 0)),
            pl.BlockSpec((1, D), lambda i: (0, 0)),
            pl.BlockSpec((D, ncols), lambda i: (0, 0)),
        ],
        out_specs=out_specs,
        out_shape=out_shape,
    )(x, agg, agg, wna, wnb0, wnb1, bn.reshape(1, D), wcat)


@functools.cache
def _sc_mesh():
    return plsc.VectorSubcoreMesh(core_axis_name="c", subcore_axis_name="s",
                                  num_cores=NC, num_subcores=NS)


def _scb_body(pv_hbm, e0_hbm, srcb_hbm, dstb_hbm, out_hbm,
              sidx0, sidx1, sidx2, sidx3, didx0, didx1, didx2, didx3,
              abuf0, abuf1, bbuf0, bbuf1, cbuf0, cbuf1,
              semi0, semi1, semi2, semi3,
              semg0, semg1, semh0, semh1, semc0, semc1):
    c = lax.axis_index("c")
    s = lax.axis_index("s")
    w = c * NS + s
    nb = NB // NW + jnp.where(w < NB % NW, 1, 0)

    abuf = (abuf0, abuf1)
    bbuf = (bbuf0, bbuf1)
    cbuf = (cbuf0, cbuf1)
    sidx = (sidx0, sidx1, sidx2, sidx3)
    didx = (didx0, didx1, didx2, didx3)
    semi = (semi0, semi1, semi2, semi3)
    semg = (semg0, semg1)
    semh = (semh0, semh1)
    semc = (semc0, semc1)

    def idx_load(i, p4):
        row = w + i * NW
        pltpu.async_copy(srcb_hbm.at[row], sidx[p4], semi[p4])
        pltpu.async_copy(dstb_hbm.at[row], didx[p4], semi[p4])

    def issue(i, p4, p):
        row = w + i * NW
        pltpu.make_async_copy(srcb_hbm.at[0], sidx[p4], semi[p4]).wait()
        pltpu.make_async_copy(srcb_hbm.at[0], didx[p4], semi[p4]).wait()
        pltpu.async_copy(pv_hbm.at[sidx[p4]], abuf[p], semg[p])
        pltpu.async_copy(pv_hbm.at[didx[p4]], bbuf[p], semh[p])
        pltpu.async_copy(e0_hbm.at[pl.ds(row * 16, 16)], cbuf[p], semc[p])

    def process(i, p4, p):
        row = w + i * NW
        pltpu.make_async_copy(pv_hbm.at[pl.ds(0, B)], abuf[p], semg[p]).wait()
        pltpu.make_async_copy(pv_hbm.at[pl.ds(0, B)], bbuf[p], semh[p]).wait()
        pltpu.make_async_copy(e0_hbm.at[pl.ds(0, 16)], cbuf[p], semc[p]).wait()

        def inner(j2, _):
            for k in range(8):
                j = j2 * 8 + k
                sl = pl.ds(16 * k, 16)
                cbuf[p][j2, sl] = jnp.maximum(
                    abuf[p][j, 0:16] + bbuf[p][j, 16:32] + cbuf[p][j2, sl],
                    0.0)
            return 0

        lax.fori_loop(0, 16, inner, 0)
        pltpu.sync_copy(cbuf[p], out_hbm.at[pl.ds(row * 16, 16)])

    idx_load(0, 0)
    idx_load(1, 1)
    issue(0, 0, 0)

    def body(i4, _):
        for k in range(4):
            i = i4 * 4 + k

            @pl.when(i + 2 < nb)
            def _():
                idx_load(i + 2, (k + 2) % 4)

            @pl.when(i + 1 < nb)
            def _():
                issue(i + 1, (k + 1) % 4, (k + 1) % 2)

            @pl.when(i < nb)
            def _():
                process(i, k, k % 2)
        return 0

    lax.fori_loop(0, (NB // NW + 4) // 4, body, 0)


@functools.cache
def _scb_kernel():
    return pl.kernel(
        _scb_body,
        out_type=jax.ShapeDtypeStruct((E // 8, 128), jnp.float32),
        mesh=_sc_mesh(),
        scratch_types=(
            [pltpu.VMEM((B,), jnp.int32)] * 8
            + [pltpu.VMEM((B, 128), jnp.float32)] * 4
            + [pltpu.VMEM((DE, 128), jnp.float32)] * 2
            + [pltpu.SemaphoreType.DMA] * 10
        ),
    )


B2 = 64
NB2 = E // B2


def _scd_body(q_hbm, t_hbm, srcb_hbm, dstb_hbm, agg_hbm,
              sidx0, sidx1, sidx2, sidx3, didx0, didx1, didx2, didx3,
              gbuf0, gbuf1, tbuf0, tbuf1, zbuf, acc,
              semi0, semi1, semi2, semi3, semg0, semg1, semt0, semt1):
    c = lax.axis_index("c")
    s = lax.axis_index("s")

    sidx = (sidx0, sidx1, sidx2, sidx3)
    didx = (didx0, didx1, didx2, didx3)
    gbuf = (gbuf0, gbuf1)
    tbuf = (tbuf0, tbuf1)
    semi = (semi0, semi1, semi2, semi3)
    semg = (semg0, semg1)
    semt = (semt0, semt1)

    zero = jnp.zeros((16,), jnp.float32)

    def zrow(j, _):
        for k in range(8):
            zbuf[j, pl.ds(16 * k, 16)] = zero
        return 0

    lax.fori_loop(0, 48, zrow, 0)
    base = s * 624
    for k in range(13):
        pltpu.sync_copy(zbuf, acc.at[pl.ds(base + 48 * k, 48)])

    @pl.when(s == NS - 1)
    def _():
        pltpu.sync_copy(zbuf.at[pl.ds(0, 16)], acc.at[pl.ds(9984, 16)])

    plsc.subcore_barrier()

    nb = NB2 // NS + jnp.where(s < NB2 % NS, 1, 0)
    qoff = c * N

    def idx_load(i, p4):
        row = s + i * NS
        pltpu.async_copy(srcb_hbm.at[row], sidx[p4], semi[p4])
        pltpu.async_copy(dstb_hbm.at[row], didx[p4], semi[p4])

    def issue(i, p4, p):
        row = s + i * NS
        pltpu.make_async_copy(srcb_hbm.at[0], sidx[p4], semi[p4]).wait()
        pltpu.make_async_copy(srcb_hbm.at[0], didx[p4], semi[p4]).wait()
        for k in range(4):
            sidx[p4][pl.ds(16 * k, 16)] = sidx[p4][pl.ds(16 * k, 16)] + qoff
        pltpu.async_copy(q_hbm.at[sidx[p4]], gbuf[p], semg[p])
        pltpu.async_copy(t_hbm.at[pl.ds(c * (E // 8) + row * 8, 8)],
                         tbuf[p], semt[p])

    def process(i, p4, p):
        pltpu.make_async_copy(q_hbm.at[pl.ds(0, B2)], gbuf[p],
                              semg[p]).wait()
        pltpu.make_async_copy(t_hbm.at[pl.ds(0, 8)], tbuf[p],
                              semt[p]).wait()

        def inner(jr, _):
            for k in range(8):
                j = jr * 8 + k
                for m in range(8):
                    sl = pl.ds(16 * m, 16)
                    slt = pl.ds(128 * k + 16 * m, 16)
                    gbuf[p][j, sl] = jnp.maximum(
                        gbuf[p][j, sl] + tbuf[p][jr, slt], 0.0)
            return 0

        lax.fori_loop(0, 8, inner, 0)
        pltpu.sync_copy(gbuf[p], acc.at[didx[p4]], add=True)

    idx_load(0, 0)
    idx_load(1, 1)
    issue(0, 0, 0)

    def body(i4, _):
        for k in range(4):
            i = i4 * 4 + k

            @pl.when(i + 2 < nb)
            def _():
                idx_load(i + 2, (k + 2) % 4)

            @pl.when(i + 1 < nb)
            def _():
                issue(i + 1, (k + 1) % 4, (k + 1) % 2)

            @pl.when(i < nb)
            def _():
                process(i, k, k % 2)
        return 0

    lax.fori_loop(0, (NB2 // NS + 4) // 4, body, 0)
    plsc.subcore_barrier()
    pltpu.sync_copy(acc.at[pl.ds(base, 624)],
                    agg_hbm.at[pl.ds(c * N + base, 624)])

    @pl.when(s == NS - 1)
    def _():
        pltpu.sync_copy(acc.at[pl.ds(9984, 16)],
                        agg_hbm.at[pl.ds(c * N + 9984, 16)])


@functools.cache
def _scd_kernel():
    return pl.kernel(
        _scd_body,
        out_type=jax.ShapeDtypeStruct((2 * N, 128), jnp.float32),
        mesh=_sc_mesh(),
        scratch_types=(
            [pltpu.VMEM((B2,), jnp.int32)] * 8
            + [pltpu.VMEM((B2, 128), jnp.float32)] * 2
            + [pltpu.VMEM((8, 1024), jnp.float32)] * 2
            + [pltpu.VMEM((48, 128), jnp.float32)]
            + [pltpu.VMEM_SHARED((N, 128), jnp.float32)]
            + [pltpu.SemaphoreType.DMA] * 8
        ),
    )


def kernel(edge_index, x, z,
           We0, be0, Wm0, bm0, Wn0, bn0,
           We1, be1, Wm1, bm1, Wn1, bn1,
           We2, be2, Wm2, bm2, Wn2, bn2):
    src = edge_index[0].astype(jnp.int32)
    dst = edge_index[1].astype(jnp.int32)
    srcb = src.reshape(NB, B)
    dstb = dst.reshape(NB, B)
    srcb2 = src.reshape(NB2, B2)
    dstb2 = dst.reshape(NB2, B2)
    x = x.astype(jnp.float32)

    wcat0 = jnp.concatenate([Wm0[:D], We0[:D], We0[D:2 * D]], axis=1)
    wcat1 = jnp.concatenate([Wm1[:D], We1[:D], We1[D:2 * D]], axis=1)
    wsd2 = jnp.concatenate([We2[:D], We2[D:2 * D]], axis=1)

    def bd8(w):
        return jsl.block_diag(*([w] * 8))

    def eprep(wee, be):
        return bd8(wee), jnp.tile(be, 8).reshape(1, 128)

    def tprep(wm, bm):
        wbig = jnp.concatenate([bd8(wm[D:, 0:128]), bd8(wm[D:, 128:256])],
                               axis=1)
        bmw = jnp.concatenate([jnp.tile(bm[0:128], 8),
                               jnp.tile(bm[128:256], 8)]).reshape(1, 2048)
        return wbig, bmw

    wee80, bew0 = eprep(We0[2 * D:], be0)
    wee81, bew1 = eprep(We1[2 * D:], be1)
    wee82, bew2 = eprep(We2[2 * D:], be2)
    wbig0, bmw0 = tprep(Wm0, bm0)
    wbig1, bmw1 = tprep(Wm1, bm1)

    def scb(pv, e0pw):
        return _scb_kernel()(pv, e0pw, srcb, dstb)

    q0, pv0 = _node_proj(x, wcat0)
    e0p = _edge_proj(z.reshape(E // 8, 128), wee80, bew0)
    e1 = scb(pv0, e0p)
    t0, e1p = _t_proj(e1, e1, wbig0, bmw0, wee81, bew1, avg=False)
    agg0 = _scd_kernel()(q0.reshape(2 * N, 128),
                         t0.reshape(2 * (E // 8), 1024), srcb2, dstb2)

    x1, q1, pv1 = _node_update(
        x, agg0.reshape(2, N, 128), Wn0[:D], Wn0[D:D + 128],
        Wn0[D + 128:], bn0, wcat1, avg=False)
    e2 = scb(pv1, e1p)
    t1, e2p = _t_proj(e2, e1, wbig1, bmw1, wee82, bew2, avg=True)
    agg1 = _scd_kernel()(q1.reshape(2 * N, 128),
                         t1.reshape(2 * (E // 8), 1024), srcb2, dstb2)

    pv2 = _node_update(
        x1, agg1.reshape(2, N, 128), Wn1[:D], Wn1[D:D + 128],
        Wn1[D + 128:], bn1, wsd2, avg=True)
    return scb(pv2, e2p).reshape(E, DE)


# speedup vs baseline: 1.4871x; 1.4871x over previous
import functools

import jax
import jax.numpy as jnp
import jax.scipy.linalg as jsl
from jax import lax
from jax.experimental import pallas as pl
from jax.experimental.pallas import tpu as pltpu
from jax.experimental.pallas import tpu_sc as plsc

N = 10000
E = 160000
D = 256
DE = 16

NC = 2
NS = 16
NW = NC * NS
B = 128
NB = E // B
RPS = N // NS


_NBLK = 1000
_EBLK = 3200


def _nodeproj_body(x_ref, w_ref, q_ref, pv_ref):
    r = jnp.dot(x_ref[...], w_ref[...], preferred_element_type=jnp.float32)
    q_ref[0, :, :] = r[:, 0:128]
    q_ref[1, :, :] = r[:, 128:256]
    pv_ref[:, 0:32] = r[:, 256:288]
    pv_ref[:, 32:128] = jnp.zeros_like(r[:, 32:128])


def _node_proj(x, wcat):
    grid = (N // _NBLK,)
    q, pv = pl.pallas_call(
        _nodeproj_body,
        grid=grid,
        in_specs=[
            pl.BlockSpec((_NBLK, D), lambda i: (i, 0)),
            pl.BlockSpec((D, 288), lambda i: (0, 0)),
        ],
        out_specs=[
            pl.BlockSpec((2, _NBLK, 128), lambda i: (0, i, 0)),
            pl.BlockSpec((_NBLK, 128), lambda i: (i, 0)),
        ],
        out_shape=[
            jax.ShapeDtypeStruct((2, N, 128), jnp.float32),
            jax.ShapeDtypeStruct((N, 128), jnp.float32),
        ],
    )(x, wcat)
    return q, pv


def _edgeproj_body(e_ref, w_ref, b_ref, o_ref):
    o_ref[...] = (
        jnp.dot(e_ref[...], w_ref[...], preferred_element_type=jnp.float32)
        + b_ref[...]
    )


def _edge_proj(eaw, wee8, bew):
    wblk = _EBLK // 8
    return pl.pallas_call(
        _edgeproj_body,
        grid=(E // _EBLK,),
        in_specs=[
            pl.BlockSpec((wblk, 128), lambda i: (i, 0)),
            pl.BlockSpec((128, 128), lambda i: (0, 0)),
            pl.BlockSpec((1, 128), lambda i: (0, 0)),
        ],
        out_specs=pl.BlockSpec((wblk, 128), lambda i: (i, 0)),
        out_shape=jax.ShapeDtypeStruct((E // 8, 128), jnp.float32),
    )(eaw, wee8, bew)


def _tproj_body(avg, e_ref, ep_ref, wbig_ref, bmw_ref, wee_ref, be_ref,
                t_ref, en_ref):
    e = e_ref[...]
    t = jnp.dot(e, wbig_ref[...], preferred_element_type=jnp.float32) + bmw_ref[...]
    for cc in range(2):
        for k in range(8):
            t_ref[cc, :, k, :] = t[:, 1024 * cc + 128 * k:
                                   1024 * cc + 128 * k + 128]
    ea = 0.5 * (e + ep_ref[...]) if avg else e
    en_ref[...] = (
        jnp.dot(ea, wee_ref[...], preferred_element_type=jnp.float32)
        + be_ref[...]
    )


def _t_proj(e_new, e_prev, wbig, bmw, wee8, bew, avg):
    wblk = _EBLK // 8
    t, en = pl.pallas_call(
        functools.partial(_tproj_body, avg),
        grid=(E // _EBLK,),
        in_specs=[
            pl.BlockSpec((wblk, 128), lambda i: (i, 0)),
            pl.BlockSpec((wblk, 128), lambda i: (i, 0)),
            pl.BlockSpec((128, 2048), lambda i: (0, 0)),
            pl.BlockSpec((1, 2048), lambda i: (0, 0)),
            pl.BlockSpec((128, 128), lambda i: (0, 0)),
            pl.BlockSpec((1, 128), lambda i: (0, 0)),
        ],
        out_specs=[
            pl.BlockSpec((2, wblk, 8, 128), lambda i: (0, i, 0, 0)),
            pl.BlockSpec((wblk, 128), lambda i: (i, 0)),
        ],
        out_shape=[
            jax.ShapeDtypeStruct((2, E // 8, 8, 128), jnp.float32),
            jax.ShapeDtypeStruct((E // 8, 128), jnp.float32),
        ],
    )(e_new, e_prev, wbig, bmw, wee8, bew)
    return t, en


def _nodeup_body(avg, ncols, x_ref, a0_ref, a1_ref, wna_ref, wnb0_ref,
                 wnb1_ref, bn_ref, wcat_ref, *out_refs):
    x = x_ref[...]
    r = jnp.dot(x, wna_ref[...], preferred_element_type=jnp.float32)
    r = r + jnp.dot(a0_ref[0], wnb0_ref[...], preferred_element_type=jnp.float32)
    r = r + jnp.dot(a1_ref[0], wnb1_ref[...], preferred_element_type=jnp.float32)
    r = jnp.maximum(r + bn_ref[...], 0.0)
    if avg:
        r = 0.5 * (x + r)
    p = jnp.dot(r, wcat_ref[...], preferred_element_type=jnp.float32)
    if ncols == 288:
        xo_ref, q_ref, pv_ref = out_refs
        xo_ref[...] = r
        q_ref[0, :, :] = p[:, 0:128]
        q_ref[1, :, :] = p[:, 128:256]
        pv_ref[:, 0:32] = p[:, 256:288]
        pv_ref[:, 32:128] = jnp.zeros_like(p[:, 32:128])
    else:
        pv_ref, = out_refs
        pv_ref[:, 0:32] = p[:, 0:32]
        pv_ref[:, 32:128] = jnp.zeros((p.shape[0], 96), jnp.float32)


def _node_update(x, agg, wna, wnb0, wnb1, bn, wcat, avg):
    ncols = wcat.shape[1]
    if ncols == 288:
        out_specs = [
            pl.BlockSpec((_NBLK, D), lambda i: (i, 0)),
            pl.BlockSpec((2, _NBLK, 128), lambda i: (0, i, 0)),
            pl.BlockSpec((_NBLK, 128), lambda i: (i, 0)),
        ]
        out_shape = [
            jax.ShapeDtypeStruct((N, D), jnp.float32),
            jax.ShapeDtypeStruct((2, N, 128), jnp.float32),
            jax.ShapeDtypeStruct((N, 128), jnp.float32),
        ]
    else:
        out_specs = pl.BlockSpec((_NBLK, 128), lambda i: (i, 0))
        out_shape = jax.ShapeDtypeStruct((N, 128), jnp.float32)
    return pl.pallas_call(
        functools.partial(_nodeup_body, avg, ncols),
        grid=(N // _NBLK,),
        in_specs=[
            pl.BlockSpec((_NBLK, D), lambda i: (i, 0)),
            pl.BlockSpec((1, _NBLK, 128), lambda i: (0, i, 0)),
            pl.BlockSpec((1, _NBLK, 128), lambda i: (1, i, 0)),
            pl.BlockSpec((D, D), lambda i: (0, 0)),
            pl.BlockSpec((128, D), lambda i: (0, 0)),
            pl.BlockSpec((128, D), lambda i: (0, 0)),
            pl.BlockSpec((1, D), lambda i: (0, 0)),
            pl.BlockSpec((D, ncols), lambda i: (0, 0)),
        ],
        out_specs=out_specs,
        out_shape=out_shape,
    )(x, agg, agg, wna, wnb0, wnb1, bn.reshape(1, D), wcat)


@functools.cache
def _sc_mesh():
    return plsc.VectorSubcoreMesh(core_axis_name="c", subcore_axis_name="s",
                                  num_cores=NC, num_subcores=NS)


def _scb_body(pv_hbm, e0_hbm, srcb_hbm, dstb_hbm, out_hbm,
              sidx0, sidx1, sidx2, sidx3, didx0, didx1, didx2, didx3,
              abuf0, abuf1, bbuf0, bbuf1, cbuf0, cbuf1,
              semi0, semi1, semi2, semi3,
              semg0, semg1, semh0, semh1, semc0, semc1):
    c = lax.axis_index("c")
    s = lax.axis_index("s")
    w = c * NS + s
    nb = NB // NW + jnp.where(w < NB % NW, 1, 0)

    abuf = (abuf0, abuf1)
    bbuf = (bbuf0, bbuf1)
    cbuf = (cbuf0, cbuf1)
    sidx = (sidx0, sidx1, sidx2, sidx3)
    didx = (didx0, didx1, didx2, didx3)
    semi = (semi0, semi1, semi2, semi3)
    semg = (semg0, semg1)
    semh = (semh0, semh1)
    semc = (semc0, semc1)

    def idx_load(i, p4):
        row = w + i * NW
        pltpu.async_copy(srcb_hbm.at[row], sidx[p4], semi[p4])
        pltpu.async_copy(dstb_hbm.at[row], didx[p4], semi[p4])

    def issue(i, p4, p):
        row = w + i * NW
        pltpu.make_async_copy(srcb_hbm.at[0], sidx[p4], semi[p4]).wait()
        pltpu.make_async_copy(srcb_hbm.at[0], didx[p4], semi[p4]).wait()
        pltpu.async_copy(pv_hbm.at[sidx[p4]], abuf[p], semg[p])
        pltpu.async_copy(pv_hbm.at[didx[p4]], bbuf[p], semh[p])
        pltpu.async_copy(e0_hbm.at[pl.ds(row * 16, 16)], cbuf[p], semc[p])

    def process(i, p4, p):
        row = w + i * NW
        pltpu.make_async_copy(pv_hbm.at[pl.ds(0, B)], abuf[p], semg[p]).wait()
        pltpu.make_async_copy(pv_hbm.at[pl.ds(0, B)], bbuf[p], semh[p]).wait()
        pltpu.make_async_copy(e0_hbm.at[pl.ds(0, 16)], cbuf[p], semc[p]).wait()

        def inner(j2, _):
            for k in range(8):
                j = j2 * 8 + k
                sl = pl.ds(16 * k, 16)
                cbuf[p][j2, sl] = jnp.maximum(
                    abuf[p][j, 0:16] + bbuf[p][j, 16:32] + cbuf[p][j2, sl],
                    0.0)
            return 0

        lax.fori_loop(0, 16, inner, 0)
        pltpu.sync_copy(cbuf[p], out_hbm.at[pl.ds(row * 16, 16)])

    idx_load(0, 0)
    idx_load(1, 1)
    issue(0, 0, 0)

    def body(i4, _):
        for k in range(4):
            i = i4 * 4 + k

            @pl.when(i + 2 < nb)
            def _():
                idx_load(i + 2, (k + 2) % 4)

            @pl.when(i + 1 < nb)
            def _():
                issue(i + 1, (k + 1) % 4, (k + 1) % 2)

            @pl.when(i < nb)
            def _():
                process(i, k, k % 2)
        return 0

    lax.fori_loop(0, (NB // NW + 4) // 4, body, 0)


@functools.cache
def _scb_kernel():
    return pl.kernel(
        _scb_body,
        out_type=jax.ShapeDtypeStruct((E // 8, 128), jnp.float32),
        mesh=_sc_mesh(),
        scratch_types=(
            [pltpu.VMEM((B,), jnp.int32)] * 8
            + [pltpu.VMEM((B, 128), jnp.float32)] * 4
            + [pltpu.VMEM((DE, 128), jnp.float32)] * 2
            + [pltpu.SemaphoreType.DMA] * 10
        ),
    )


B2 = 64
NB2 = E // B2


def _scd_body(q_hbm, t_hbm, srcb_hbm, dstb_hbm, agg_hbm,
              sidx0, sidx1, sidx2, sidx3, didx0, didx1, didx2, didx3,
              gbuf0, gbuf1, tbuf0, tbuf1, zbuf, acc,
              semi0, semi1, semi2, semi3, semg0, semg1, semt0, semt1):
    c = lax.axis_index("c")
    s = lax.axis_index("s")

    sidx = (sidx0, sidx1, sidx2, sidx3)
    didx = (didx0, didx1, didx2, didx3)
    gbuf = (gbuf0, gbuf1)
    tbuf = (tbuf0, tbuf1)
    semi = (semi0, semi1, semi2, semi3)
    semg = (semg0, semg1)
    semt = (semt0, semt1)

    zero = jnp.zeros((16,), jnp.float32)

    def zrow(j, _):
        for k in range(8):
            zbuf[j, pl.ds(16 * k, 16)] = zero
        return 0

    lax.fori_loop(0, 48, zrow, 0)
    base = s * 624
    for k in range(13):
        pltpu.sync_copy(zbuf, acc.at[pl.ds(base + 48 * k, 48)])

    @pl.when(s == NS - 1)
    def _():
        pltpu.sync_copy(zbuf.at[pl.ds(0, 16)], acc.at[pl.ds(9984, 16)])

    plsc.subcore_barrier()

    nb = NB2 // NS + jnp.where(s < NB2 % NS, 1, 0)
    qoff = c * N

    def idx_load(i, p4):
        row = s + i * NS
        pltpu.async_copy(srcb_hbm.at[row], sidx[p4], semi[p4])
        pltpu.async_copy(dstb_hbm.at[row], didx[p4], semi[p4])

    def issue(i, p4, p):
        row = s + i * NS
        pltpu.make_async_copy(srcb_hbm.at[0], sidx[p4], semi[p4]).wait()
        pltpu.make_async_copy(srcb_hbm.at[0], didx[p4], semi[p4]).wait()
        for k in range(4):
            sidx[p4][pl.ds(16 * k, 16)] = sidx[p4][pl.ds(16 * k, 16)] + qoff
        pltpu.async_copy(q_hbm.at[sidx[p4]], gbuf[p], semg[p])
        pltpu.async_copy(t_hbm.at[pl.ds(c * E + row * B2, B2)], tbuf[p],
                         semt[p])

    def process(i, p4, p):
        pltpu.make_async_copy(q_hbm.at[pl.ds(0, B2)], gbuf[p],
                              semg[p]).wait()
        pltpu.make_async_copy(t_hbm.at[pl.ds(0, B2)], tbuf[p],
                              semt[p]).wait()

        def inner(j, _):
            for k in range(8):
                sl = pl.ds(16 * k, 16)
                gbuf[p][j, sl] = jnp.maximum(
                    gbuf[p][j, sl] + tbuf[p][j, sl], 0.0)
            return 0

        lax.fori_loop(0, B2, inner, 0)
        pltpu.sync_copy(gbuf[p], acc.at[didx[p4]], add=True)

    idx_load(0, 0)
    idx_load(1, 1)
    issue(0, 0, 0)

    def body(i4, _):
        for k in range(4):
            i = i4 * 4 + k

            @pl.when(i + 2 < nb)
            def _():
                idx_load(i + 2, (k + 2) % 4)

            @pl.when(i + 1 < nb)
            def _():
                issue(i + 1, (k + 1) % 4, (k + 1) % 2)

            @pl.when(i < nb)
            def _():
                process(i, k, k % 2)
        return 0

    lax.fori_loop(0, (NB2 // NS + 4) // 4, body, 0)
    plsc.subcore_barrier()
    pltpu.sync_copy(acc.at[pl.ds(base, 624)],
                    agg_hbm.at[pl.ds(c * N + base, 624)])

    @pl.when(s == NS - 1)
    def _():
        pltpu.sync_copy(acc.at[pl.ds(9984, 16)],
                        agg_hbm.at[pl.ds(c * N + 9984, 16)])


@functools.cache
def _scd_kernel():
    return pl.kernel(
        _scd_body,
        out_type=jax.ShapeDtypeStruct((2 * N, 128), jnp.float32),
        mesh=_sc_mesh(),
        scratch_types=(
            [pltpu.VMEM((B2,), jnp.int32)] * 8
            + [pltpu.VMEM((B2, 128), jnp.float32)] * 4
            + [pltpu.VMEM((48, 128), jnp.float32)]
            + [pltpu.VMEM_SHARED((N, 128), jnp.float32)]
            + [pltpu.SemaphoreType.DMA] * 8
        ),
    )


def kernel(edge_index, x, z,
           We0, be0, Wm0, bm0, Wn0, bn0,
           We1, be1, Wm1, bm1, Wn1, bn1,
           We2, be2, Wm2, bm2, Wn2, bn2):
    src = edge_index[0].astype(jnp.int32)
    dst = edge_index[1].astype(jnp.int32)
    srcb = src.reshape(NB, B)
    dstb = dst.reshape(NB, B)
    srcb2 = src.reshape(NB2, B2)
    dstb2 = dst.reshape(NB2, B2)
    x = x.astype(jnp.float32)

    wcat0 = jnp.concatenate([Wm0[:D], We0[:D], We0[D:2 * D]], axis=1)
    wcat1 = jnp.concatenate([Wm1[:D], We1[:D], We1[D:2 * D]], axis=1)
    wsd2 = jnp.concatenate([We2[:D], We2[D:2 * D]], axis=1)

    def bd8(w):
        return jsl.block_diag(*([w] * 8))

    def eprep(wee, be):
        return bd8(wee), jnp.tile(be, 8).reshape(1, 128)

    def tprep(wm, bm):
        wbig = jnp.concatenate([bd8(wm[D:, 0:128]), bd8(wm[D:, 128:256])],
                               axis=1)
        bmw = jnp.concatenate([jnp.tile(bm[0:128], 8),
                               jnp.tile(bm[128:256], 8)]).reshape(1, 2048)
        return wbig, bmw

    wee80, bew0 = eprep(We0[2 * D:], be0)
    wee81, bew1 = eprep(We1[2 * D:], be1)
    wee82, bew2 = eprep(We2[2 * D:], be2)
    wbig0, bmw0 = tprep(Wm0, bm0)
    wbig1, bmw1 = tprep(Wm1, bm1)

    def scb(pv, e0pw):
        return _scb_kernel()(pv, e0pw, srcb, dstb)

    q0, pv0 = _node_proj(x, wcat0)
    e0p = _edge_proj(z.reshape(E // 8, 128), wee80, bew0)
    e1 = scb(pv0, e0p)
    t0, e1p = _t_proj(e1, e1, wbig0, bmw0, wee81, bew1, avg=False)
    agg0 = _scd_kernel()(q0.reshape(2 * N, 128),
                         t0.reshape(2 * E, 128), srcb2, dstb2)

    x1, q1, pv1 = _node_update(
        x, agg0.reshape(2, N, 128), Wn0[:D], Wn0[D:D + 128],
        Wn0[D + 128:], bn0, wcat1, avg=False)
    e2 = scb(pv1, e1p)
    t1, e2p = _t_proj(e2, e1, wbig1, bmw1, wee82, bew2, avg=True)
    agg1 = _scd_kernel()(q1.reshape(2 * N, 128),
                         t1.reshape(2 * E, 128), srcb2, dstb2)

    pv2 = _node_update(
        x1, agg1.reshape(2, N, 128), Wn1[:D], Wn1[D:D + 128],
        Wn1[D + 128:], bn1, wsd2, avg=True)
    return scb(pv2, e2p).reshape(E, DE)


# speedup vs baseline: 1.5595x; 1.0487x over previous
import functools

import jax
import jax.numpy as jnp
import jax.scipy.linalg as jsl
from jax import lax
from jax.experimental import pallas as pl
from jax.experimental.pallas import tpu as pltpu
from jax.experimental.pallas import tpu_sc as plsc

N = 10000
E = 160000
D = 256
DE = 16

NC = 2
NS = 16
NW = NC * NS
B = 128
NB = E // B
RPS = N // NS


_NBLK = 1000
_EBLK = 3200


def _nodeproj_body(x_ref, w_ref, q_ref, pv_ref):
    r = jnp.dot(x_ref[...], w_ref[...], preferred_element_type=jnp.float32)
    q_ref[0, :, :] = r[:, 0:128]
    q_ref[1, :, :] = r[:, 128:256]
    pv_ref[:, 0:32] = r[:, 256:288]
    pv_ref[:, 32:128] = jnp.zeros_like(r[:, 32:128])


def _node_proj(x, wcat):
    grid = (N // _NBLK,)
    q, pv = pl.pallas_call(
        _nodeproj_body,
        grid=grid,
        in_specs=[
            pl.BlockSpec((_NBLK, D), lambda i: (i, 0)),
            pl.BlockSpec((D, 288), lambda i: (0, 0)),
        ],
        out_specs=[
            pl.BlockSpec((2, _NBLK, 128), lambda i: (0, i, 0)),
            pl.BlockSpec((_NBLK, 128), lambda i: (i, 0)),
        ],
        out_shape=[
            jax.ShapeDtypeStruct((2, N, 128), jnp.float32),
            jax.ShapeDtypeStruct((N, 128), jnp.float32),
        ],
    )(x, wcat)
    return q, pv


def _edgeproj_body(e_ref, w_ref, b_ref, o_ref):
    o_ref[...] = (
        jnp.dot(e_ref[...], w_ref[...], preferred_element_type=jnp.float32)
        + b_ref[...]
    )


def _edge_proj(eaw, wee8, bew):
    wblk = _EBLK // 8
    return pl.pallas_call(
        _edgeproj_body,
        grid=(E // _EBLK,),
        in_specs=[
            pl.BlockSpec((wblk, 128), lambda i: (i, 0)),
            pl.BlockSpec((128, 128), lambda i: (0, 0)),
            pl.BlockSpec((1, 128), lambda i: (0, 0)),
        ],
        out_specs=pl.BlockSpec((wblk, 128), lambda i: (i, 0)),
        out_shape=jax.ShapeDtypeStruct((E // 8, 128), jnp.float32),
    )(eaw, wee8, bew)


def _tproj_body(avg, e_ref, ep_ref, wbig_ref, bmw_ref, wee_ref, be_ref,
                t_ref, en_ref):
    e = e_ref[...]
    t = jnp.dot(e, wbig_ref[...], preferred_element_type=jnp.float32) + bmw_ref[...]
    for cc in range(2):
        for k in range(8):
            t_ref[cc, :, k, :] = t[:, 1024 * cc + 128 * k:
                                   1024 * cc + 128 * k + 128]
    ea = 0.5 * (e + ep_ref[...]) if avg else e
    en_ref[...] = (
        jnp.dot(ea, wee_ref[...], preferred_element_type=jnp.float32)
        + be_ref[...]
    )


def _t_proj(e_new, e_prev, wbig, bmw, wee8, bew, avg):
    wblk = _EBLK // 8
    t, en = pl.pallas_call(
        functools.partial(_tproj_body, avg),
        grid=(E // _EBLK,),
        in_specs=[
            pl.BlockSpec((wblk, 128), lambda i: (i, 0)),
            pl.BlockSpec((wblk, 128), lambda i: (i, 0)),
            pl.BlockSpec((128, 2048), lambda i: (0, 0)),
            pl.BlockSpec((1, 2048), lambda i: (0, 0)),
            pl.BlockSpec((128, 128), lambda i: (0, 0)),
            pl.BlockSpec((1, 128), lambda i: (0, 0)),
        ],
        out_specs=[
            pl.BlockSpec((2, wblk, 8, 128), lambda i: (0, i, 0, 0)),
            pl.BlockSpec((wblk, 128), lambda i: (i, 0)),
        ],
        out_shape=[
            jax.ShapeDtypeStruct((2, E // 8, 8, 128), jnp.float32),
            jax.ShapeDtypeStruct((E // 8, 128), jnp.float32),
        ],
    )(e_new, e_prev, wbig, bmw, wee8, bew)
    return t, en


def _nodeup_body(avg, ncols, x_ref, a0_ref, a1_ref, wna_ref, wnb0_ref,
                 wnb1_ref, bn_ref, wcat_ref, *out_refs):
    x = x_ref[...]
    r = jnp.dot(x, wna_ref[...], preferred_element_type=jnp.float32)
    r = r + jnp.dot(a0_ref[0], wnb0_ref[...], preferred_element_type=jnp.float32)
    r = r + jnp.dot(a1_ref[0], wnb1_ref[...], preferred_element_type=jnp.float32)
    r = jnp.maximum(r + bn_ref[...], 0.0)
    if avg:
        r = 0.5 * (x + r)
    p = jnp.dot(r, wcat_ref[...], preferred_element_type=jnp.float32)
    if ncols == 288:
        xo_ref, q_ref, pv_ref = out_refs
        xo_ref[...] = r
        q_ref[0, :, :] = p[:, 0:128]
        q_ref[1, :, :] = p[:, 128:256]
        pv_ref[:, 0:32] = p[:, 256:288]
        pv_ref[:, 32:128] = jnp.zeros_like(p[:, 32:128])
    else:
        pv_ref, = out_refs
        pv_ref[:, 0:32] = p[:, 0:32]
        pv_ref[:, 32:128] = jnp.zeros((p.shape[0], 96), jnp.float32)


def _node_update(x, agg, wna, wnb0, wnb1, bn, wcat, avg):
    ncols = wcat.shape[1]
    if ncols == 288:
        out_specs = [
            pl.BlockSpec((_NBLK, D), lambda i: (i, 0)),
            pl.BlockSpec((2, _NBLK, 128), lambda i: (0, i, 0)),
            pl.BlockSpec((_NBLK, 128), lambda i: (i, 0)),
        ]
        out_shape = [
            jax.ShapeDtypeStruct((N, D), jnp.float32),
            jax.ShapeDtypeStruct((2, N, 128), jnp.float32),
            jax.ShapeDtypeStruct((N, 128), jnp.float32),
        ]
    else:
        out_specs = pl.BlockSpec((_NBLK, 128), lambda i: (i, 0))
        out_shape = jax.ShapeDtypeStruct((N, 128), jnp.float32)
    return pl.pallas_call(
        functools.partial(_nodeup_body, avg, ncols),
        grid=(N // _NBLK,),
        in_specs=[
            pl.BlockSpec((_NBLK, D), lambda i: (i, 0)),
            pl.BlockSpec((1, _NBLK, 128), lambda i: (0, i, 0)),
            pl.BlockSpec((1, _NBLK, 128), lambda i: (1, i, 0)),
            pl.BlockSpec((D, D), lambda i: (0, 0)),
            pl.BlockSpec((128, D), lambda i: (0, 0)),
            pl.BlockSpec((128, D), lambda i: (0, 0)),
            pl.BlockSpec((1, D), lambda i: (0, 0)),
            pl.BlockSpec((D, ncols), lambda i: (0, 0)),
        ],
        out_specs=out_specs,
        out_shape=out_shape,
    )(x, agg, agg, wna, wnb0, wnb1, bn.reshape(1, D), wcat)


@functools.cache
def _sc_mesh():
    return plsc.VectorSubcoreMesh(core_axis_name="c", subcore_axis_name="s",
                                  num_cores=NC, num_subcores=NS)


def _scb_body(pv_hbm, e0_hbm, srcb_hbm, dstb_hbm, out_hbm,
              sidx0, sidx1, sidx2, sidx3, didx0, didx1, didx2, didx3,
              abuf0, abuf1, bbuf0, bbuf1, cbuf0, cbuf1, pvs,
              semi0, semi1, semi2, semi3,
              semg0, semg1, semh0, semh1, semc0, semc1):
    c = lax.axis_index("c")
    s = lax.axis_index("s")
    w = c * NS + s
    nb = NB2 // NW + jnp.where(w < NB2 % NW, 1, 0)

    sbase = s * 624
    pltpu.sync_copy(pv_hbm.at[pl.ds(sbase, 624)], pvs.at[pl.ds(sbase, 624)])

    @pl.when(s == NS - 1)
    def _():
        pltpu.sync_copy(pv_hbm.at[pl.ds(9984, 16)], pvs.at[pl.ds(9984, 16)])

    plsc.subcore_barrier()

    abuf = (abuf0, abuf1)
    bbuf = (bbuf0, bbuf1)
    cbuf = (cbuf0, cbuf1)
    sidx = (sidx0, sidx1, sidx2, sidx3)
    didx = (didx0, didx1, didx2, didx3)
    semi = (semi0, semi1, semi2, semi3)
    semg = (semg0, semg1)
    semh = (semh0, semh1)
    semc = (semc0, semc1)

    def idx_load(i, p4):
        row = w + i * NW
        pltpu.async_copy(srcb_hbm.at[row], sidx[p4], semi[p4])
        pltpu.async_copy(dstb_hbm.at[row], didx[p4], semi[p4])

    def issue(i, p4, p):
        row = w + i * NW
        pltpu.make_async_copy(srcb_hbm.at[0], sidx[p4], semi[p4]).wait()
        pltpu.make_async_copy(srcb_hbm.at[0], didx[p4], semi[p4]).wait()
        pltpu.async_copy(pvs.at[sidx[p4]], abuf[p], semg[p])
        pltpu.async_copy(pvs.at[didx[p4]], bbuf[p], semh[p])
        pltpu.async_copy(e0_hbm.at[pl.ds(row * 8, 8)], cbuf[p], semc[p])

    def process(i, p4, p):
        row = w + i * NW
        pltpu.make_async_copy(pvs.at[pl.ds(0, B2)], abuf[p], semg[p]).wait()
        pltpu.make_async_copy(pvs.at[pl.ds(0, B2)], bbuf[p], semh[p]).wait()
        pltpu.make_async_copy(e0_hbm.at[pl.ds(0, 8)], cbuf[p], semc[p]).wait()

        def inner(j2, _):
            for k in range(8):
                j = j2 * 8 + k
                sl = pl.ds(16 * k, 16)
                cbuf[p][j2, sl] = jnp.maximum(
                    abuf[p][j, 0:16] + bbuf[p][j, 16:32] + cbuf[p][j2, sl],
                    0.0)
            return 0

        lax.fori_loop(0, 8, inner, 0)
        pltpu.sync_copy(cbuf[p], out_hbm.at[pl.ds(row * 8, 8)])

    idx_load(0, 0)
    idx_load(1, 1)
    issue(0, 0, 0)

    def body(i4, _):
        for k in range(4):
            i = i4 * 4 + k

            @pl.when(i + 2 < nb)
            def _():
                idx_load(i + 2, (k + 2) % 4)

            @pl.when(i + 1 < nb)
            def _():
                issue(i + 1, (k + 1) % 4, (k + 1) % 2)

            @pl.when(i < nb)
            def _():
                process(i, k, k % 2)
        return 0

    lax.fori_loop(0, (NB2 // NW + 4) // 4, body, 0)


@functools.cache
def _scb_kernel():
    return pl.kernel(
        _scb_body,
        out_type=jax.ShapeDtypeStruct((E // 8, 128), jnp.float32),
        mesh=_sc_mesh(),
        scratch_types=(
            [pltpu.VMEM((B2,), jnp.int32)] * 8
            + [pltpu.VMEM((B2, 128), jnp.float32)] * 4
            + [pltpu.VMEM((8, 128), jnp.float32)] * 2
            + [pltpu.VMEM_SHARED((N, 128), jnp.float32)]
            + [pltpu.SemaphoreType.DMA] * 10
        ),
    )


B2 = 64
NB2 = E // B2


def _scd_body(q_hbm, t_hbm, srcb_hbm, dstb_hbm, agg_hbm,
              sidx0, sidx1, sidx2, sidx3, didx0, didx1, didx2, didx3,
              gbuf0, gbuf1, tbuf0, tbuf1, zbuf, acc,
              semi0, semi1, semi2, semi3, semg0, semg1, semt0, semt1):
    c = lax.axis_index("c")
    s = lax.axis_index("s")

    sidx = (sidx0, sidx1, sidx2, sidx3)
    didx = (didx0, didx1, didx2, didx3)
    gbuf = (gbuf0, gbuf1)
    tbuf = (tbuf0, tbuf1)
    semi = (semi0, semi1, semi2, semi3)
    semg = (semg0, semg1)
    semt = (semt0, semt1)

    zero = jnp.zeros((16,), jnp.float32)

    def zrow(j, _):
        for k in range(8):
            zbuf[j, pl.ds(16 * k, 16)] = zero
        return 0

    lax.fori_loop(0, 48, zrow, 0)
    base = s * 624
    for k in range(13):
        pltpu.sync_copy(zbuf, acc.at[pl.ds(base + 48 * k, 48)])

    @pl.when(s == NS - 1)
    def _():
        pltpu.sync_copy(zbuf.at[pl.ds(0, 16)], acc.at[pl.ds(9984, 16)])

    plsc.subcore_barrier()

    nb = NB2 // NS + jnp.where(s < NB2 % NS, 1, 0)
    qoff = c * N

    def idx_load(i, p4):
        row = s + i * NS
        pltpu.async_copy(srcb_hbm.at[row], sidx[p4], semi[p4])
        pltpu.async_copy(dstb_hbm.at[row], didx[p4], semi[p4])

    def issue(i, p4, p):
        row = s + i * NS
        pltpu.make_async_copy(srcb_hbm.at[0], sidx[p4], semi[p4]).wait()
        pltpu.make_async_copy(srcb_hbm.at[0], didx[p4], semi[p4]).wait()
        for k in range(4):
            sidx[p4][pl.ds(16 * k, 16)] = sidx[p4][pl.ds(16 * k, 16)] + qoff
        pltpu.async_copy(q_hbm.at[sidx[p4]], gbuf[p], semg[p])
        pltpu.async_copy(t_hbm.at[pl.ds(c * E + row * B2, B2)], tbuf[p],
                         semt[p])

    def process(i, p4, p):
        pltpu.make_async_copy(q_hbm.at[pl.ds(0, B2)], gbuf[p],
                              semg[p]).wait()
        pltpu.make_async_copy(t_hbm.at[pl.ds(0, B2)], tbuf[p],
                              semt[p]).wait()

        def inner(j, _):
            for k in range(8):
                sl = pl.ds(16 * k, 16)
                gbuf[p][j, sl] = jnp.maximum(
                    gbuf[p][j, sl] + tbuf[p][j, sl], 0.0)
            return 0

        lax.fori_loop(0, B2, inner, 0)
        pltpu.sync_copy(gbuf[p], acc.at[didx[p4]], add=True)

    idx_load(0, 0)
    idx_load(1, 1)
    issue(0, 0, 0)

    def body(i4, _):
        for k in range(4):
            i = i4 * 4 + k

            @pl.when(i + 2 < nb)
            def _():
                idx_load(i + 2, (k + 2) % 4)

            @pl.when(i + 1 < nb)
            def _():
                issue(i + 1, (k + 1) % 4, (k + 1) % 2)

            @pl.when(i < nb)
            def _():
                process(i, k, k % 2)
        return 0

    lax.fori_loop(0, (NB2 // NS + 4) // 4, body, 0)
    plsc.subcore_barrier()
    pltpu.sync_copy(acc.at[pl.ds(base, 624)],
                    agg_hbm.at[pl.ds(c * N + base, 624)])

    @pl.when(s == NS - 1)
    def _():
        pltpu.sync_copy(acc.at[pl.ds(9984, 16)],
                        agg_hbm.at[pl.ds(c * N + 9984, 16)])


@functools.cache
def _scd_kernel():
    return pl.kernel(
        _scd_body,
        out_type=jax.ShapeDtypeStruct((2 * N, 128), jnp.float32),
        mesh=_sc_mesh(),
        scratch_types=(
            [pltpu.VMEM((B2,), jnp.int32)] * 8
            + [pltpu.VMEM((B2, 128), jnp.float32)] * 4
            + [pltpu.VMEM((48, 128), jnp.float32)]
            + [pltpu.VMEM_SHARED((N, 128), jnp.float32)]
            + [pltpu.SemaphoreType.DMA] * 8
        ),
    )


def kernel(edge_index, x, z,
           We0, be0, Wm0, bm0, Wn0, bn0,
           We1, be1, Wm1, bm1, Wn1, bn1,
           We2, be2, Wm2, bm2, Wn2, bn2):
    src = edge_index[0].astype(jnp.int32)
    dst = edge_index[1].astype(jnp.int32)
    srcb = src.reshape(NB, B)
    dstb = dst.reshape(NB, B)
    srcb2 = src.reshape(NB2, B2)
    dstb2 = dst.reshape(NB2, B2)
    x = x.astype(jnp.float32)

    wcat0 = jnp.concatenate([Wm0[:D], We0[:D], We0[D:2 * D]], axis=1)
    wcat1 = jnp.concatenate([Wm1[:D], We1[:D], We1[D:2 * D]], axis=1)
    wsd2 = jnp.concatenate([We2[:D], We2[D:2 * D]], axis=1)

    def bd8(w):
        return jsl.block_diag(*([w] * 8))

    def eprep(wee, be):
        return bd8(wee), jnp.tile(be, 8).reshape(1, 128)

    def tprep(wm, bm):
        wbig = jnp.concatenate([bd8(wm[D:, 0:128]), bd8(wm[D:, 128:256])],
                               axis=1)
        bmw = jnp.concatenate([jnp.tile(bm[0:128], 8),
                               jnp.tile(bm[128:256], 8)]).reshape(1, 2048)
        return wbig, bmw

    wee80, bew0 = eprep(We0[2 * D:], be0)
    wee81, bew1 = eprep(We1[2 * D:], be1)
    wee82, bew2 = eprep(We2[2 * D:], be2)
    wbig0, bmw0 = tprep(Wm0, bm0)
    wbig1, bmw1 = tprep(Wm1, bm1)

    def scb(pv, e0pw):
        return _scb_kernel()(pv, e0pw, srcb2, dstb2)

    q0, pv0 = _node_proj(x, wcat0)
    e0p = _edge_proj(z.reshape(E // 8, 128), wee80, bew0)
    e1 = scb(pv0, e0p)
    t0, e1p = _t_proj(e1, e1, wbig0, bmw0, wee81, bew1, avg=False)
    agg0 = _scd_kernel()(q0.reshape(2 * N, 128),
                         t0.reshape(2 * E, 128), srcb2, dstb2)

    x1, q1, pv1 = _node_update(
        x, agg0.reshape(2, N, 128), Wn0[:D], Wn0[D:D + 128],
        Wn0[D + 128:], bn0, wcat1, avg=False)
    e2 = scb(pv1, e1p)
    t1, e2p = _t_proj(e2, e1, wbig1, bmw1, wee82, bew2, avg=True)
    agg1 = _scd_kernel()(q1.reshape(2 * N, 128),
                         t1.reshape(2 * E, 128), srcb2, dstb2)

    pv2 = _node_update(
        x1, agg1.reshape(2, N, 128), Wn1[:D], Wn1[D:D + 128],
        Wn1[D + 128:], bn1, wsd2, avg=True)
    return scb(pv2, e2p).reshape(E, DE)


# speedup vs baseline: 1.5916x; 1.0206x over previous
import functools

import jax
import jax.numpy as jnp
import jax.scipy.linalg as jsl
from jax import lax
from jax.experimental import pallas as pl
from jax.experimental.pallas import tpu as pltpu
from jax.experimental.pallas import tpu_sc as plsc

N = 10000
E = 160000
D = 256
DE = 16

NC = 2
NS = 16
NW = NC * NS
B = 128
NB = E // B
RPS = N // NS


_NBLK = 1000
_EBLK = 6400


def _nodeproj_body(x_ref, w_ref, q_ref, pv_ref):
    r = jnp.dot(x_ref[...], w_ref[...], preferred_element_type=jnp.float32)
    q_ref[0, :, :] = r[:, 0:128]
    q_ref[1, :, :] = r[:, 128:256]
    pv_ref[:, 0:32] = r[:, 256:288]
    pv_ref[:, 32:128] = jnp.zeros_like(r[:, 32:128])


def _node_proj(x, wcat):
    grid = (N // _NBLK,)
    q, pv = pl.pallas_call(
        _nodeproj_body,
        grid=grid,
        in_specs=[
            pl.BlockSpec((_NBLK, D), lambda i: (i, 0)),
            pl.BlockSpec((D, 288), lambda i: (0, 0)),
        ],
        out_specs=[
            pl.BlockSpec((2, _NBLK, 128), lambda i: (0, i, 0)),
            pl.BlockSpec((_NBLK, 128), lambda i: (i, 0)),
        ],
        out_shape=[
            jax.ShapeDtypeStruct((2, N, 128), jnp.float32),
            jax.ShapeDtypeStruct((N, 128), jnp.float32),
        ],
    )(x, wcat)
    return q, pv


def _edgeproj_body(e_ref, w_ref, b_ref, o_ref):
    o_ref[...] = (
        jnp.dot(e_ref[...], w_ref[...], preferred_element_type=jnp.float32)
        + b_ref[...]
    )


def _edge_proj(eaw, wee8, bew):
    wblk = _EBLK // 8
    return pl.pallas_call(
        _edgeproj_body,
        grid=(E // _EBLK,),
        in_specs=[
            pl.BlockSpec((wblk, 128), lambda i: (i, 0)),
            pl.BlockSpec((128, 128), lambda i: (0, 0)),
            pl.BlockSpec((1, 128), lambda i: (0, 0)),
        ],
        out_specs=pl.BlockSpec((wblk, 128), lambda i: (i, 0)),
        out_shape=jax.ShapeDtypeStruct((E // 8, 128), jnp.float32),
    )(eaw, wee8, bew)


def _tproj_body(avg, e_ref, ep_ref, wbig_ref, bmw_ref, wee_ref, be_ref,
                t_ref, en_ref):
    e = e_ref[...]
    t = jnp.dot(e, wbig_ref[...], preferred_element_type=jnp.float32) + bmw_ref[...]
    for cc in range(2):
        for k in range(8):
            t_ref[cc, :, k, :] = t[:, 1024 * cc + 128 * k:
                                   1024 * cc + 128 * k + 128]
    ea = 0.5 * (e + ep_ref[...]) if avg else e
    en_ref[...] = (
        jnp.dot(ea, wee_ref[...], preferred_element_type=jnp.float32)
        + be_ref[...]
    )


def _t_proj(e_new, e_prev, wbig, bmw, wee8, bew, avg):
    wblk = _EBLK // 8
    t, en = pl.pallas_call(
        functools.partial(_tproj_body, avg),
        grid=(E // _EBLK,),
        in_specs=[
            pl.BlockSpec((wblk, 128), lambda i: (i, 0)),
            pl.BlockSpec((wblk, 128), lambda i: (i, 0)),
            pl.BlockSpec((128, 2048), lambda i: (0, 0)),
            pl.BlockSpec((1, 2048), lambda i: (0, 0)),
            pl.BlockSpec((128, 128), lambda i: (0, 0)),
            pl.BlockSpec((1, 128), lambda i: (0, 0)),
        ],
        out_specs=[
            pl.BlockSpec((2, wblk, 8, 128), lambda i: (0, i, 0, 0)),
            pl.BlockSpec((wblk, 128), lambda i: (i, 0)),
        ],
        out_shape=[
            jax.ShapeDtypeStruct((2, E // 8, 8, 128), jnp.float32),
            jax.ShapeDtypeStruct((E // 8, 128), jnp.float32),
        ],
    )(e_new, e_prev, wbig, bmw, wee8, bew)
    return t, en


def _nodeup_body(avg, ncols, x_ref, a0_ref, a1_ref, wna_ref, wnb0_ref,
                 wnb1_ref, bn_ref, wcat_ref, *out_refs):
    x = x_ref[...]
    r = jnp.dot(x, wna_ref[...], preferred_element_type=jnp.float32)
    r = r + jnp.dot(a0_ref[0], wnb0_ref[...], preferred_element_type=jnp.float32)
    r = r + jnp.dot(a1_ref[0], wnb1_ref[...], preferred_element_type=jnp.float32)
    r = jnp.maximum(r + bn_ref[...], 0.0)
    if avg:
        r = 0.5 * (x + r)
    p = jnp.dot(r, wcat_ref[...], preferred_element_type=jnp.float32)
    if ncols == 288:
        xo_ref, q_ref, pv_ref = out_refs
        xo_ref[...] = r
        q_ref[0, :, :] = p[:, 0:128]
        q_ref[1, :, :] = p[:, 128:256]
        pv_ref[:, 0:32] = p[:, 256:288]
        pv_ref[:, 32:128] = jnp.zeros_like(p[:, 32:128])
    else:
        pv_ref, = out_refs
        pv_ref[:, 0:32] = p[:, 0:32]
        pv_ref[:, 32:128] = jnp.zeros((p.shape[0], 96), jnp.float32)


def _node_update(x, agg, wna, wnb0, wnb1, bn, wcat, avg):
    ncols = wcat.shape[1]
    if ncols == 288:
        out_specs = [
            pl.BlockSpec((_NBLK, D), lambda i: (i, 0)),
            pl.BlockSpec((2, _NBLK, 128), lambda i: (0, i, 0)),
            pl.BlockSpec((_NBLK, 128), lambda i: (i, 0)),
        ]
        out_shape = [
            jax.ShapeDtypeStruct((N, D), jnp.float32),
            jax.ShapeDtypeStruct((2, N, 128), jnp.float32),
            jax.ShapeDtypeStruct((N, 128), jnp.float32),
        ]
    else:
        out_specs = pl.BlockSpec((_NBLK, 128), lambda i: (i, 0))
        out_shape = jax.ShapeDtypeStruct((N, 128), jnp.float32)
    return pl.pallas_call(
        functools.partial(_nodeup_body, avg, ncols),
        grid=(N // _NBLK,),
        in_specs=[
            pl.BlockSpec((_NBLK, D), lambda i: (i, 0)),
            pl.BlockSpec((1, _NBLK, 128), lambda i: (0, i, 0)),
            pl.BlockSpec((1, _NBLK, 128), lambda i: (1, i, 0)),
            pl.BlockSpec((D, D), lambda i: (0, 0)),
            pl.BlockSpec((128, D), lambda i: (0, 0)),
            pl.BlockSpec((128, D), lambda i: (0, 0)),
            pl.BlockSpec((1, D), lambda i: (0, 0)),
            pl.BlockSpec((D, ncols), lambda i: (0, 0)),
        ],
        out_specs=out_specs,
        out_shape=out_shape,
    )(x, agg, agg, wna, wnb0, wnb1, bn.reshape(1, D), wcat)


@functools.cache
def _sc_mesh():
    return plsc.VectorSubcoreMesh(core_axis_name="c", subcore_axis_name="s",
                                  num_cores=NC, num_subcores=NS)


def _scb_body(pv_hbm, e0_hbm, srcb_hbm, dstb_hbm, out_hbm,
              sidx0, sidx1, sidx2, sidx3, didx0, didx1, didx2, didx3,
              abuf0, abuf1, bbuf0, bbuf1, cbuf0, cbuf1, pvs,
              semi0, semi1, semi2, semi3,
              semg0, semg1, semh0, semh1, semc0, semc1):
    c = lax.axis_index("c")
    s = lax.axis_index("s")
    w = c * NS + s
    nb = NB2 // NW + jnp.where(w < NB2 % NW, 1, 0)

    sbase = s * 624
    pltpu.sync_copy(pv_hbm.at[pl.ds(sbase, 624)], pvs.at[pl.ds(sbase, 624)])

    @pl.when(s == NS - 1)
    def _():
        pltpu.sync_copy(pv_hbm.at[pl.ds(9984, 16)], pvs.at[pl.ds(9984, 16)])

    plsc.subcore_barrier()

    abuf = (abuf0, abuf1)
    bbuf = (bbuf0, bbuf1)
    cbuf = (cbuf0, cbuf1)
    sidx = (sidx0, sidx1, sidx2, sidx3)
    didx = (didx0, didx1, didx2, didx3)
    semi = (semi0, semi1, semi2, semi3)
    semg = (semg0, semg1)
    semh = (semh0, semh1)
    semc = (semc0, semc1)

    def idx_load(i, p4):
        row = w + i * NW
        pltpu.async_copy(srcb_hbm.at[row], sidx[p4], semi[p4])
        pltpu.async_copy(dstb_hbm.at[row], didx[p4], semi[p4])

    def issue(i, p4, p):
        row = w + i * NW
        pltpu.make_async_copy(srcb_hbm.at[0], sidx[p4], semi[p4]).wait()
        pltpu.make_async_copy(srcb_hbm.at[0], didx[p4], semi[p4]).wait()
        pltpu.async_copy(pvs.at[sidx[p4]], abuf[p], semg[p])
        pltpu.async_copy(pvs.at[didx[p4]], bbuf[p], semh[p])
        pltpu.async_copy(e0_hbm.at[pl.ds(row * 8, 8)], cbuf[p], semc[p])

    def process(i, p4, p):
        row = w + i * NW
        pltpu.make_async_copy(pvs.at[pl.ds(0, B2)], abuf[p], semg[p]).wait()
        pltpu.make_async_copy(pvs.at[pl.ds(0, B2)], bbuf[p], semh[p]).wait()
        pltpu.make_async_copy(e0_hbm.at[pl.ds(0, 8)], cbuf[p], semc[p]).wait()

        def inner(j2, _):
            for k in range(8):
                j = j2 * 8 + k
                sl = pl.ds(16 * k, 16)
                cbuf[p][j2, sl] = jnp.maximum(
                    abuf[p][j, 0:16] + bbuf[p][j, 16:32] + cbuf[p][j2, sl],
                    0.0)
            return 0

        lax.fori_loop(0, 8, inner, 0)
        pltpu.sync_copy(cbuf[p], out_hbm.at[pl.ds(row * 8, 8)])

    idx_load(0, 0)
    idx_load(1, 1)
    issue(0, 0, 0)

    def body(i4, _):
        for k in range(4):
            i = i4 * 4 + k

            @pl.when(i + 2 < nb)
            def _():
                idx_load(i + 2, (k + 2) % 4)

            @pl.when(i + 1 < nb)
            def _():
                issue(i + 1, (k + 1) % 4, (k + 1) % 2)

            @pl.when(i < nb)
            def _():
                process(i, k, k % 2)
        return 0

    lax.fori_loop(0, (NB2 // NW + 4) // 4, body, 0)


@functools.cache
def _scb_kernel():
    return pl.kernel(
        _scb_body,
        out_type=jax.ShapeDtypeStruct((E // 8, 128), jnp.float32),
        mesh=_sc_mesh(),
        scratch_types=(
            [pltpu.VMEM((B2,), jnp.int32)] * 8
            + [pltpu.VMEM((B2, 128), jnp.float32)] * 4
            + [pltpu.VMEM((8, 128), jnp.float32)] * 2
            + [pltpu.VMEM_SHARED((N, 128), jnp.float32)]
            + [pltpu.SemaphoreType.DMA] * 10
        ),
    )


B2 = 64
NB2 = E // B2


def _scd_body(q_hbm, t_hbm, srcb_hbm, dstb_hbm, agg_hbm,
              sidx0, sidx1, sidx2, sidx3, didx0, didx1, didx2, didx3,
              gbuf0, gbuf1, tbuf0, tbuf1, zbuf, acc,
              semi0, semi1, semi2, semi3, semg0, semg1, semt0, semt1):
    c = lax.axis_index("c")
    s = lax.axis_index("s")

    sidx = (sidx0, sidx1, sidx2, sidx3)
    didx = (didx0, didx1, didx2, didx3)
    gbuf = (gbuf0, gbuf1)
    tbuf = (tbuf0, tbuf1)
    semi = (semi0, semi1, semi2, semi3)
    semg = (semg0, semg1)
    semt = (semt0, semt1)

    zero = jnp.zeros((16,), jnp.float32)

    def zrow(j, _):
        for k in range(8):
            zbuf[j, pl.ds(16 * k, 16)] = zero
        return 0

    lax.fori_loop(0, 48, zrow, 0)
    base = s * 624
    for k in range(13):
        pltpu.sync_copy(zbuf, acc.at[pl.ds(base + 48 * k, 48)])

    @pl.when(s == NS - 1)
    def _():
        pltpu.sync_copy(zbuf.at[pl.ds(0, 16)], acc.at[pl.ds(9984, 16)])

    plsc.subcore_barrier()

    nb = NB2 // NS + jnp.where(s < NB2 % NS, 1, 0)
    qoff = c * N

    def idx_load(i, p4):
        row = s + i * NS
        pltpu.async_copy(srcb_hbm.at[row], sidx[p4], semi[p4])
        pltpu.async_copy(dstb_hbm.at[row], didx[p4], semi[p4])

    def issue(i, p4, p):
        row = s + i * NS
        pltpu.make_async_copy(srcb_hbm.at[0], sidx[p4], semi[p4]).wait()
        pltpu.make_async_copy(srcb_hbm.at[0], didx[p4], semi[p4]).wait()
        for k in range(4):
            sidx[p4][pl.ds(16 * k, 16)] = sidx[p4][pl.ds(16 * k, 16)] + qoff
        pltpu.async_copy(q_hbm.at[sidx[p4]], gbuf[p], semg[p])
        pltpu.async_copy(t_hbm.at[pl.ds(c * E + row * B2, B2)], tbuf[p],
                         semt[p])

    def process(i, p4, p):
        pltpu.make_async_copy(q_hbm.at[pl.ds(0, B2)], gbuf[p],
                              semg[p]).wait()
        pltpu.make_async_copy(t_hbm.at[pl.ds(0, B2)], tbuf[p],
                              semt[p]).wait()

        def inner(j, _):
            for k in range(8):
                sl = pl.ds(16 * k, 16)
                gbuf[p][j, sl] = jnp.maximum(
                    gbuf[p][j, sl] + tbuf[p][j, sl], 0.0)
            return 0

        lax.fori_loop(0, B2, inner, 0)
        pltpu.sync_copy(gbuf[p], acc.at[didx[p4]], add=True)

    idx_load(0, 0)
    idx_load(1, 1)
    issue(0, 0, 0)

    def body(i4, _):
        for k in range(4):
            i = i4 * 4 + k

            @pl.when(i + 2 < nb)
            def _():
                idx_load(i + 2, (k + 2) % 4)

            @pl.when(i + 1 < nb)
            def _():
                issue(i + 1, (k + 1) % 4, (k + 1) % 2)

            @pl.when(i < nb)
            def _():
                process(i, k, k % 2)
        return 0

    lax.fori_loop(0, (NB2 // NS + 4) // 4, body, 0)
    plsc.subcore_barrier()
    pltpu.sync_copy(acc.at[pl.ds(base, 624)],
                    agg_hbm.at[pl.ds(c * N + base, 624)])

    @pl.when(s == NS - 1)
    def _():
        pltpu.sync_copy(acc.at[pl.ds(9984, 16)],
                        agg_hbm.at[pl.ds(c * N + 9984, 16)])


@functools.cache
def _scd_kernel():
    return pl.kernel(
        _scd_body,
        out_type=jax.ShapeDtypeStruct((2 * N, 128), jnp.float32),
        mesh=_sc_mesh(),
        scratch_types=(
            [pltpu.VMEM((B2,), jnp.int32)] * 8
            + [pltpu.VMEM((B2, 128), jnp.float32)] * 4
            + [pltpu.VMEM((48, 128), jnp.float32)]
            + [pltpu.VMEM_SHARED((N, 128), jnp.float32)]
            + [pltpu.SemaphoreType.DMA] * 8
        ),
    )


def kernel(edge_index, x, z,
           We0, be0, Wm0, bm0, Wn0, bn0,
           We1, be1, Wm1, bm1, Wn1, bn1,
           We2, be2, Wm2, bm2, Wn2, bn2):
    src = edge_index[0].astype(jnp.int32)
    dst = edge_index[1].astype(jnp.int32)
    srcb = src.reshape(NB, B)
    dstb = dst.reshape(NB, B)
    srcb2 = src.reshape(NB2, B2)
    dstb2 = dst.reshape(NB2, B2)
    x = x.astype(jnp.float32)

    wcat0 = jnp.concatenate([Wm0[:D], We0[:D], We0[D:2 * D]], axis=1)
    wcat1 = jnp.concatenate([Wm1[:D], We1[:D], We1[D:2 * D]], axis=1)
    wsd2 = jnp.concatenate([We2[:D], We2[D:2 * D]], axis=1)

    def bd8(w):
        return jsl.block_diag(*([w] * 8))

    def eprep(wee, be):
        return bd8(wee), jnp.tile(be, 8).reshape(1, 128)

    def tprep(wm, bm):
        wbig = jnp.concatenate([bd8(wm[D:, 0:128]), bd8(wm[D:, 128:256])],
                               axis=1)
        bmw = jnp.concatenate([jnp.tile(bm[0:128], 8),
                               jnp.tile(bm[128:256], 8)]).reshape(1, 2048)
        return wbig, bmw

    wee80, bew0 = eprep(We0[2 * D:], be0)
    wee81, bew1 = eprep(We1[2 * D:], be1)
    wee82, bew2 = eprep(We2[2 * D:], be2)
    wbig0, bmw0 = tprep(Wm0, bm0)
    wbig1, bmw1 = tprep(Wm1, bm1)

    def scb(pv, e0pw):
        return _scb_kernel()(pv, e0pw, srcb2, dstb2)

    q0, pv0 = _node_proj(x, wcat0)
    e0p = _edge_proj(z.reshape(E // 8, 128), wee80, bew0)
    e1 = scb(pv0, e0p)
    t0, e1p = _t_proj(e1, e1, wbig0, bmw0, wee81, bew1, avg=False)
    agg0 = _scd_kernel()(q0.reshape(2 * N, 128),
                         t0.reshape(2 * E, 128), srcb2, dstb2)

    x1, q1, pv1 = _node_update(
        x, agg0.reshape(2, N, 128), Wn0[:D], Wn0[D:D + 128],
        Wn0[D + 128:], bn0, wcat1, avg=False)
    e2 = scb(pv1, e1p)
    t1, e2p = _t_proj(e2, e1, wbig1, bmw1, wee82, bew2, avg=True)
    agg1 = _scd_kernel()(q1.reshape(2 * N, 128),
                         t1.reshape(2 * E, 128), srcb2, dstb2)

    pv2 = _node_update(
        x1, agg1.reshape(2, N, 128), Wn1[:D], Wn1[D:D + 128],
        Wn1[D + 128:], bn1, wsd2, avg=True)
    return scb(pv2, e2p).reshape(E, DE)


# speedup vs baseline: 1.6070x; 1.0097x over previous
import functools

import jax
import jax.numpy as jnp
import jax.scipy.linalg as jsl
from jax import lax
from jax.experimental import pallas as pl
from jax.experimental.pallas import tpu as pltpu
from jax.experimental.pallas import tpu_sc as plsc

N = 10000
E = 160000
D = 256
DE = 16

NC = 2
NS = 16
NW = NC * NS
B = 128
NB = E // B
RPS = N // NS


_NBLK = 2000
_EBLK = 8000


def _nodeproj_body(x_ref, w_ref, q_ref, pv_ref):
    r = jnp.dot(x_ref[...], w_ref[...], preferred_element_type=jnp.float32)
    q_ref[0, :, :] = r[:, 0:128]
    q_ref[1, :, :] = r[:, 128:256]
    pv_ref[:, 0:32] = r[:, 256:288]
    pv_ref[:, 32:128] = jnp.zeros_like(r[:, 32:128])


def _node_proj(x, wcat):
    grid = (N // _NBLK,)
    q, pv = pl.pallas_call(
        _nodeproj_body,
        grid=grid,
        in_specs=[
            pl.BlockSpec((_NBLK, D), lambda i: (i, 0)),
            pl.BlockSpec((D, 288), lambda i: (0, 0)),
        ],
        out_specs=[
            pl.BlockSpec((2, _NBLK, 128), lambda i: (0, i, 0)),
            pl.BlockSpec((_NBLK, 128), lambda i: (i, 0)),
        ],
        out_shape=[
            jax.ShapeDtypeStruct((2, N, 128), jnp.float32),
            jax.ShapeDtypeStruct((N, 128), jnp.float32),
        ],
    )(x, wcat)
    return q, pv


def _edgeproj_body(e_ref, w_ref, b_ref, o_ref):
    o_ref[...] = (
        jnp.dot(e_ref[...], w_ref[...], preferred_element_type=jnp.float32)
        + b_ref[...]
    )


def _edge_proj(eaw, wee8, bew):
    wblk = _EBLK // 8
    return pl.pallas_call(
        _edgeproj_body,
        grid=(E // _EBLK,),
        in_specs=[
            pl.BlockSpec((wblk, 128), lambda i: (i, 0)),
            pl.BlockSpec((128, 128), lambda i: (0, 0)),
            pl.BlockSpec((1, 128), lambda i: (0, 0)),
        ],
        out_specs=pl.BlockSpec((wblk, 128), lambda i: (i, 0)),
        out_shape=jax.ShapeDtypeStruct((E // 8, 128), jnp.float32),
    )(eaw, wee8, bew)


def _tproj_body(avg, e_ref, ep_ref, wbig_ref, bmw_ref, wee_ref, be_ref,
                t_ref, en_ref):
    e = e_ref[...]
    t = jnp.dot(e, wbig_ref[...], preferred_element_type=jnp.float32) + bmw_ref[...]
    for cc in range(2):
        for k in range(8):
            t_ref[cc, :, k, :] = t[:, 1024 * cc + 128 * k:
                                   1024 * cc + 128 * k + 128]
    ea = 0.5 * (e + ep_ref[...]) if avg else e
    en_ref[...] = (
        jnp.dot(ea, wee_ref[...], preferred_element_type=jnp.float32)
        + be_ref[...]
    )


def _t_proj(e_new, e_prev, wbig, bmw, wee8, bew, avg):
    wblk = _EBLK // 8
    t, en = pl.pallas_call(
        functools.partial(_tproj_body, avg),
        grid=(E // _EBLK,),
        in_specs=[
            pl.BlockSpec((wblk, 128), lambda i: (i, 0)),
            pl.BlockSpec((wblk, 128), lambda i: (i, 0)),
            pl.BlockSpec((128, 2048), lambda i: (0, 0)),
            pl.BlockSpec((1, 2048), lambda i: (0, 0)),
            pl.BlockSpec((128, 128), lambda i: (0, 0)),
            pl.BlockSpec((1, 128), lambda i: (0, 0)),
        ],
        out_specs=[
            pl.BlockSpec((2, wblk, 8, 128), lambda i: (0, i, 0, 0)),
            pl.BlockSpec((wblk, 128), lambda i: (i, 0)),
        ],
        out_shape=[
            jax.ShapeDtypeStruct((2, E // 8, 8, 128), jnp.float32),
            jax.ShapeDtypeStruct((E // 8, 128), jnp.float32),
        ],
    )(e_new, e_prev, wbig, bmw, wee8, bew)
    return t, en


def _nodeup_body(avg, ncols, x_ref, a0_ref, a1_ref, wna_ref, wnb0_ref,
                 wnb1_ref, bn_ref, wcat_ref, *out_refs):
    x = x_ref[...]
    r = jnp.dot(x, wna_ref[...], preferred_element_type=jnp.float32)
    r = r + jnp.dot(a0_ref[0], wnb0_ref[...], preferred_element_type=jnp.float32)
    r = r + jnp.dot(a1_ref[0], wnb1_ref[...], preferred_element_type=jnp.float32)
    r = jnp.maximum(r + bn_ref[...], 0.0)
    if avg:
        r = 0.5 * (x + r)
    p = jnp.dot(r, wcat_ref[...], preferred_element_type=jnp.float32)
    if ncols == 288:
        xo_ref, q_ref, pv_ref = out_refs
        xo_ref[...] = r
        q_ref[0, :, :] = p[:, 0:128]
        q_ref[1, :, :] = p[:, 128:256]
        pv_ref[:, 0:32] = p[:, 256:288]
        pv_ref[:, 32:128] = jnp.zeros_like(p[:, 32:128])
    else:
        pv_ref, = out_refs
        pv_ref[:, 0:32] = p[:, 0:32]
        pv_ref[:, 32:128] = jnp.zeros((p.shape[0], 96), jnp.float32)


def _node_update(x, agg, wna, wnb0, wnb1, bn, wcat, avg):
    ncols = wcat.shape[1]
    if ncols == 288:
        out_specs = [
            pl.BlockSpec((_NBLK, D), lambda i: (i, 0)),
            pl.BlockSpec((2, _NBLK, 128), lambda i: (0, i, 0)),
            pl.BlockSpec((_NBLK, 128), lambda i: (i, 0)),
        ]
        out_shape = [
            jax.ShapeDtypeStruct((N, D), jnp.float32),
            jax.ShapeDtypeStruct((2, N, 128), jnp.float32),
            jax.ShapeDtypeStruct((N, 128), jnp.float32),
        ]
    else:
        out_specs = pl.BlockSpec((_NBLK, 128), lambda i: (i, 0))
        out_shape = jax.ShapeDtypeStruct((N, 128), jnp.float32)
    return pl.pallas_call(
        functools.partial(_nodeup_body, avg, ncols),
        grid=(N // _NBLK,),
        in_specs=[
            pl.BlockSpec((_NBLK, D), lambda i: (i, 0)),
            pl.BlockSpec((1, _NBLK, 128), lambda i: (0, i, 0)),
            pl.BlockSpec((1, _NBLK, 128), lambda i: (1, i, 0)),
            pl.BlockSpec((D, D), lambda i: (0, 0)),
            pl.BlockSpec((128, D), lambda i: (0, 0)),
            pl.BlockSpec((128, D), lambda i: (0, 0)),
            pl.BlockSpec((1, D), lambda i: (0, 0)),
            pl.BlockSpec((D, ncols), lambda i: (0, 0)),
        ],
        out_specs=out_specs,
        out_shape=out_shape,
    )(x, agg, agg, wna, wnb0, wnb1, bn.reshape(1, D), wcat)


@functools.cache
def _sc_mesh():
    return plsc.VectorSubcoreMesh(core_axis_name="c", subcore_axis_name="s",
                                  num_cores=NC, num_subcores=NS)


def _scb_body(pv_hbm, e0_hbm, srcb_hbm, dstb_hbm, out_hbm,
              sidx0, sidx1, sidx2, sidx3, didx0, didx1, didx2, didx3,
              abuf0, abuf1, bbuf0, bbuf1, cbuf0, cbuf1, pvs,
              semi0, semi1, semi2, semi3,
              semg0, semg1, semh0, semh1, semc0, semc1):
    c = lax.axis_index("c")
    s = lax.axis_index("s")
    w = c * NS + s
    nb = NB2 // NW + jnp.where(w < NB2 % NW, 1, 0)

    sbase = s * 624
    pltpu.sync_copy(pv_hbm.at[pl.ds(sbase, 624)], pvs.at[pl.ds(sbase, 624)])

    @pl.when(s == NS - 1)
    def _():
        pltpu.sync_copy(pv_hbm.at[pl.ds(9984, 16)], pvs.at[pl.ds(9984, 16)])

    plsc.subcore_barrier()

    abuf = (abuf0, abuf1)
    bbuf = (bbuf0, bbuf1)
    cbuf = (cbuf0, cbuf1)
    sidx = (sidx0, sidx1, sidx2, sidx3)
    didx = (didx0, didx1, didx2, didx3)
    semi = (semi0, semi1, semi2, semi3)
    semg = (semg0, semg1)
    semh = (semh0, semh1)
    semc = (semc0, semc1)

    def idx_load(i, p4):
        row = w + i * NW
        pltpu.async_copy(srcb_hbm.at[row], sidx[p4], semi[p4])
        pltpu.async_copy(dstb_hbm.at[row], didx[p4], semi[p4])

    def issue(i, p4, p):
        row = w + i * NW
        pltpu.make_async_copy(srcb_hbm.at[0], sidx[p4], semi[p4]).wait()
        pltpu.make_async_copy(srcb_hbm.at[0], didx[p4], semi[p4]).wait()
        pltpu.async_copy(pvs.at[sidx[p4]], abuf[p], semg[p])
        pltpu.async_copy(pvs.at[didx[p4]], bbuf[p], semh[p])
        pltpu.async_copy(e0_hbm.at[pl.ds(row * 8, 8)], cbuf[p], semc[p])

    def process(i, p4, p):
        row = w + i * NW
        pltpu.make_async_copy(pvs.at[pl.ds(0, B2)], abuf[p], semg[p]).wait()
        pltpu.make_async_copy(pvs.at[pl.ds(0, B2)], bbuf[p], semh[p]).wait()
        pltpu.make_async_copy(e0_hbm.at[pl.ds(0, 8)], cbuf[p], semc[p]).wait()

        def inner(j2, _):
            for k in range(8):
                j = j2 * 8 + k
                sl = pl.ds(16 * k, 16)
                cbuf[p][j2, sl] = jnp.maximum(
                    abuf[p][j, 0:16] + bbuf[p][j, 16:32] + cbuf[p][j2, sl],
                    0.0)
            return 0

        lax.fori_loop(0, 8, inner, 0)
        pltpu.sync_copy(cbuf[p], out_hbm.at[pl.ds(row * 8, 8)])

    idx_load(0, 0)
    idx_load(1, 1)
    issue(0, 0, 0)

    def body(i4, _):
        for k in range(4):
            i = i4 * 4 + k

            @pl.when(i + 2 < nb)
            def _():
                idx_load(i + 2, (k + 2) % 4)

            @pl.when(i + 1 < nb)
            def _():
                issue(i + 1, (k + 1) % 4, (k + 1) % 2)

            @pl.when(i < nb)
            def _():
                process(i, k, k % 2)
        return 0

    lax.fori_loop(0, (NB2 // NW + 4) // 4, body, 0)


@functools.cache
def _scb_kernel():
    return pl.kernel(
        _scb_body,
        out_type=jax.ShapeDtypeStruct((E // 8, 128), jnp.float32),
        mesh=_sc_mesh(),
        scratch_types=(
            [pltpu.VMEM((B2,), jnp.int32)] * 8
            + [pltpu.VMEM((B2, 128), jnp.float32)] * 4
            + [pltpu.VMEM((8, 128), jnp.float32)] * 2
            + [pltpu.VMEM_SHARED((N, 128), jnp.float32)]
            + [pltpu.SemaphoreType.DMA] * 10
        ),
    )


B2 = 64
NB2 = E // B2


def _scd_body(q_hbm, t_hbm, srcb_hbm, dstb_hbm, agg_hbm,
              sidx0, sidx1, sidx2, sidx3, didx0, didx1, didx2, didx3,
              gbuf0, gbuf1, tbuf0, tbuf1, zbuf, acc,
              semi0, semi1, semi2, semi3, semg0, semg1, semt0, semt1):
    c = lax.axis_index("c")
    s = lax.axis_index("s")

    sidx = (sidx0, sidx1, sidx2, sidx3)
    didx = (didx0, didx1, didx2, didx3)
    gbuf = (gbuf0, gbuf1)
    tbuf = (tbuf0, tbuf1)
    semi = (semi0, semi1, semi2, semi3)
    semg = (semg0, semg1)
    semt = (semt0, semt1)

    zero = jnp.zeros((16,), jnp.float32)

    def zrow(j, _):
        for k in range(8):
            zbuf[j, pl.ds(16 * k, 16)] = zero
        return 0

    lax.fori_loop(0, 48, zrow, 0)
    base = s * 624
    for k in range(13):
        pltpu.sync_copy(zbuf, acc.at[pl.ds(base + 48 * k, 48)])

    @pl.when(s == NS - 1)
    def _():
        pltpu.sync_copy(zbuf.at[pl.ds(0, 16)], acc.at[pl.ds(9984, 16)])

    plsc.subcore_barrier()

    nb = NB2 // NS + jnp.where(s < NB2 % NS, 1, 0)
    qoff = c * N

    def idx_load(i, p4):
        row = s + i * NS
        pltpu.async_copy(srcb_hbm.at[row], sidx[p4], semi[p4])
        pltpu.async_copy(dstb_hbm.at[row], didx[p4], semi[p4])

    def issue(i, p4, p):
        row = s + i * NS
        pltpu.make_async_copy(srcb_hbm.at[0], sidx[p4], semi[p4]).wait()
        pltpu.make_async_copy(srcb_hbm.at[0], didx[p4], semi[p4]).wait()
        for k in range(4):
            sidx[p4][pl.ds(16 * k, 16)] = sidx[p4][pl.ds(16 * k, 16)] + qoff
        pltpu.async_copy(q_hbm.at[sidx[p4]], gbuf[p], semg[p])
        pltpu.async_copy(t_hbm.at[pl.ds(c * E + row * B2, B2)], tbuf[p],
                         semt[p])

    def process(i, p4, p):
        pltpu.make_async_copy(q_hbm.at[pl.ds(0, B2)], gbuf[p],
                              semg[p]).wait()
        pltpu.make_async_copy(t_hbm.at[pl.ds(0, B2)], tbuf[p],
                              semt[p]).wait()

        def inner(j, _):
            for k in range(8):
                sl = pl.ds(16 * k, 16)
                gbuf[p][j, sl] = jnp.maximum(
                    gbuf[p][j, sl] + tbuf[p][j, sl], 0.0)
            return 0

        lax.fori_loop(0, B2, inner, 0)
        pltpu.sync_copy(gbuf[p], acc.at[didx[p4]], add=True)

    idx_load(0, 0)
    idx_load(1, 1)
    issue(0, 0, 0)

    def body(i4, _):
        for k in range(4):
            i = i4 * 4 + k

            @pl.when(i + 2 < nb)
            def _():
                idx_load(i + 2, (k + 2) % 4)

            @pl.when(i + 1 < nb)
            def _():
                issue(i + 1, (k + 1) % 4, (k + 1) % 2)

            @pl.when(i < nb)
            def _():
                process(i, k, k % 2)
        return 0

    lax.fori_loop(0, (NB2 // NS + 4) // 4, body, 0)
    plsc.subcore_barrier()
    pltpu.sync_copy(acc.at[pl.ds(base, 624)],
                    agg_hbm.at[pl.ds(c * N + base, 624)])

    @pl.when(s == NS - 1)
    def _():
        pltpu.sync_copy(acc.at[pl.ds(9984, 16)],
                        agg_hbm.at[pl.ds(c * N + 9984, 16)])


@functools.cache
def _scd_kernel():
    return pl.kernel(
        _scd_body,
        out_type=jax.ShapeDtypeStruct((2 * N, 128), jnp.float32),
        mesh=_sc_mesh(),
        scratch_types=(
            [pltpu.VMEM((B2,), jnp.int32)] * 8
            + [pltpu.VMEM((B2, 128), jnp.float32)] * 4
            + [pltpu.VMEM((48, 128), jnp.float32)]
            + [pltpu.VMEM_SHARED((N, 128), jnp.float32)]
            + [pltpu.SemaphoreType.DMA] * 8
        ),
    )


def kernel(edge_index, x, z,
           We0, be0, Wm0, bm0, Wn0, bn0,
           We1, be1, Wm1, bm1, Wn1, bn1,
           We2, be2, Wm2, bm2, Wn2, bn2):
    src = edge_index[0].astype(jnp.int32)
    dst = edge_index[1].astype(jnp.int32)
    srcb = src.reshape(NB, B)
    dstb = dst.reshape(NB, B)
    srcb2 = src.reshape(NB2, B2)
    dstb2 = dst.reshape(NB2, B2)
    x = x.astype(jnp.float32)

    wcat0 = jnp.concatenate([Wm0[:D], We0[:D], We0[D:2 * D]], axis=1)
    wcat1 = jnp.concatenate([Wm1[:D], We1[:D], We1[D:2 * D]], axis=1)
    wsd2 = jnp.concatenate([We2[:D], We2[D:2 * D]], axis=1)

    def bd8(w):
        return jsl.block_diag(*([w] * 8))

    def eprep(wee, be):
        return bd8(wee), jnp.tile(be, 8).reshape(1, 128)

    def tprep(wm, bm):
        wbig = jnp.concatenate([bd8(wm[D:, 0:128]), bd8(wm[D:, 128:256])],
                               axis=1)
        bmw = jnp.concatenate([jnp.tile(bm[0:128], 8),
                               jnp.tile(bm[128:256], 8)]).reshape(1, 2048)
        return wbig, bmw

    wee80, bew0 = eprep(We0[2 * D:], be0)
    wee81, bew1 = eprep(We1[2 * D:], be1)
    wee82, bew2 = eprep(We2[2 * D:], be2)
    wbig0, bmw0 = tprep(Wm0, bm0)
    wbig1, bmw1 = tprep(Wm1, bm1)

    def scb(pv, e0pw):
        return _scb_kernel()(pv, e0pw, srcb2, dstb2)

    q0, pv0 = _node_proj(x, wcat0)
    e0p = _edge_proj(z.reshape(E // 8, 128), wee80, bew0)
    e1 = scb(pv0, e0p)
    t0, e1p = _t_proj(e1, e1, wbig0, bmw0, wee81, bew1, avg=False)
    agg0 = _scd_kernel()(q0.reshape(2 * N, 128),
                         t0.reshape(2 * E, 128), srcb2, dstb2)

    x1, q1, pv1 = _node_update(
        x, agg0.reshape(2, N, 128), Wn0[:D], Wn0[D:D + 128],
        Wn0[D + 128:], bn0, wcat1, avg=False)
    e2 = scb(pv1, e1p)
    t1, e2p = _t_proj(e2, e1, wbig1, bmw1, wee82, bew2, avg=True)
    agg1 = _scd_kernel()(q1.reshape(2 * N, 128),
                         t1.reshape(2 * E, 128), srcb2, dstb2)

    pv2 = _node_update(
        x1, agg1.reshape(2, N, 128), Wn1[:D], Wn1[D:D + 128],
        Wn1[D + 128:], bn1, wsd2, avg=True)
    return scb(pv2, e2p).reshape(E, DE)
